# baseline (speedup 1.0000x reference)
import functools

import jax
import jax.numpy as jnp
from jax.experimental import pallas as pl
from jax.experimental.pallas import tpu as pltpu


def _gate_kernel(x_ref, gw_ref, bias_ref, combine_ref):
    x = x_ref[...]
    gw = gw_ref[...]
    t, e = combine_ref.shape
    logits = jax.lax.dot_general(
        x.astype(jnp.bfloat16), gw.astype(jnp.bfloat16), (((1,), (1,)), ((), ())),
        preferred_element_type=jnp.float32,
    )
    scores = jax.nn.sigmoid(logits)
    sfc = scores + bias_ref[...]
    eidx = jax.lax.broadcasted_iota(jnp.int32, (t, e), 1)
    m1 = jnp.max(sfc, axis=1, keepdims=True)
    i1 = jnp.min(jnp.where(sfc == m1, eidx, e), axis=1, keepdims=True)
    oh1 = eidx == i1
    w1 = jnp.sum(jnp.where(oh1, scores, 0.0), axis=1, keepdims=True)
    sfc2 = jnp.where(oh1, -1e30, sfc)
    m2 = jnp.max(sfc2, axis=1, keepdims=True)
    i2 = jnp.min(jnp.where(sfc2 == m2, eidx, e), axis=1, keepdims=True)
    oh2 = eidx == i2
    w2 = jnp.sum(jnp.where(oh2, scores, 0.0), axis=1, keepdims=True)
    denom = w1 + w2 + 1e-20
    combine_ref[...] = (jnp.where(oh1, w1, 0.0) + jnp.where(oh2, w2, 0.0)) / denom


def _mlp_kernel(x_ref, w1e_ref, w2e_ref, ws1_ref, ws2_ref, c_ref, out_ref, *, n_routed):
    j = pl.program_id(1)
    x = x_ref[...]

    def expert_contrib():
        a = jax.lax.dot_general(x, w1e_ref[0], (((1,), (1,)), ((), ())),
                                preferred_element_type=jnp.float32)
        h = jnp.square(jnp.maximum(a, 0.0)).astype(jnp.bfloat16)
        y = jax.lax.dot_general(h, w2e_ref[0], (((1,), (1,)), ((), ())),
                                preferred_element_type=jnp.float32)
        c = c_ref[...]
        eidx = jax.lax.broadcasted_iota(jnp.int32, c.shape, 1)
        cj = jnp.sum(jnp.where(eidx == j, c, 0.0), axis=1, keepdims=True)
        return cj * y

    def shared_contrib():
        a = jax.lax.dot_general(x, ws1_ref[0], (((1,), (1,)), ((), ())),
                                preferred_element_type=jnp.float32)
        h = jnp.square(jnp.maximum(a, 0.0)).astype(jnp.bfloat16)
        y = jax.lax.dot_general(h, ws2_ref[...], (((1,), (1,)), ((), ())),
                                preferred_element_type=jnp.float32)
        return y

    contrib = jnp.where(j < n_routed, expert_contrib(), shared_contrib())

    @pl.when(j == 0)
    def _():
        out_ref[...] = contrib

    @pl.when(j > 0)
    def _():
        out_ref[...] += contrib


def kernel(hidden_states, gate_weight, e_score_correction_bias, shared_w1,
           shared_w2, expert_w1, expert_w2):
    T, H = hidden_states.shape
    E, I_, _ = expert_w1.shape
    SI = shared_w1.shape[0]
    n_sh = SI // I_
    NE = E + n_sh
    TB = min(512, T)

    x = hidden_states
    combine = pl.pallas_call(
        _gate_kernel,
        out_shape=jax.ShapeDtypeStruct((T, E), jnp.float32),
    )(x, gate_weight, e_score_correction_bias.reshape(1, E))

    x_bf = x.astype(jnp.bfloat16)
    w1e = expert_w1.astype(jnp.bfloat16)
    w2e = expert_w2.astype(jnp.bfloat16)
    ws1 = shared_w1.reshape(n_sh, I_, H).astype(jnp.bfloat16)
    ws2 = shared_w2.astype(jnp.bfloat16)

    grid = (T // TB, NE)

    out = pl.pallas_call(
        functools.partial(_mlp_kernel, n_routed=E),
        grid=grid,
        in_specs=[
            pl.BlockSpec((TB, H), lambda i, j: (i, 0)),
            pl.BlockSpec((1, I_, H), lambda i, j: (jnp.minimum(j, E - 1), 0, 0)),
            pl.BlockSpec((1, H, I_), lambda i, j: (jnp.minimum(j, E - 1), 0, 0)),
            pl.BlockSpec((1, I_, H), lambda i, j: (jnp.maximum(j - E, 0), 0, 0)),
            pl.BlockSpec((H, I_), lambda i, j: (0, jnp.maximum(j - E, 0))),
            pl.BlockSpec((TB, E), lambda i, j: (i, 0)),
        ],
        out_specs=pl.BlockSpec((TB, H), lambda i, j: (i, 0)),
        out_shape=jax.ShapeDtypeStruct((T, H), jnp.float32),
        compiler_params=pltpu.CompilerParams(
            dimension_semantics=("parallel", "arbitrary"),
        ),
    )(x_bf, w1e, w2e, ws1, ws2, combine)
    return out


# speedup vs baseline: 1.3229x; 1.3229x over previous
import functools

import jax
import jax.numpy as jnp
from jax import lax
from jax.experimental import pallas as pl
from jax.experimental.pallas import tpu as pltpu
from jax.experimental.pallas import tpu_sc as plsc

_B = 128
_CHUNK = 64
_CCH = 32


def _route_kernel(x_ref, gw_ref, bias_ref, pos_ref, w_ref, bexp_ref, nact_ref,
                  *, n_blocks):
    x = x_ref[...]
    gw = gw_ref[...]
    t, e = x.shape[0], gw.shape[0]
    logits = lax.dot_general(
        x.astype(jnp.bfloat16), gw.astype(jnp.bfloat16), (((1,), (1,)), ((), ())),
        preferred_element_type=jnp.float32)
    scores = jax.nn.sigmoid(logits)
    sfc = scores + bias_ref[...]
    eidx = lax.broadcasted_iota(jnp.int32, (t, e), 1)
    m1 = jnp.max(sfc, axis=1, keepdims=True)
    i1 = jnp.min(jnp.where(sfc == m1, eidx, e), axis=1, keepdims=True)
    oh1 = eidx == i1
    w1 = jnp.sum(jnp.where(oh1, scores, 0.0), axis=1, keepdims=True)
    sfc2 = jnp.where(oh1, -1e30, sfc)
    m2 = jnp.max(sfc2, axis=1, keepdims=True)
    i2 = jnp.min(jnp.where(sfc2 == m2, eidx, e), axis=1, keepdims=True)
    oh2 = eidx == i2
    w2 = jnp.sum(jnp.where(oh2, scores, 0.0), axis=1, keepdims=True)
    denom = w1 + w2 + 1e-20

    oh = (oh1 | oh2).astype(jnp.float32)
    C = 256
    lt = (lax.broadcasted_iota(jnp.int32, (C, C), 0)
          > lax.broadcasted_iota(jnp.int32, (C, C), 1)).astype(jnp.float32)
    run = jnp.zeros((1, e), jnp.float32)
    cums = []
    for c in range(t // C):
        ohc = oh[c * C:(c + 1) * C]
        exc = lax.dot_general(lt, ohc, (((1,), (0,)), ((), ())),
                              preferred_element_type=jnp.float32) + run
        cums.append(exc)
        run = run + jnp.sum(ohc, axis=0, keepdims=True)
    cum = jnp.concatenate(cums, axis=0)
    counts = run

    bf = jnp.float32(_B)
    nblk_row = jnp.floor((counts + (bf - 1.0)) / bf)
    m_le = (lax.broadcasted_iota(jnp.int32, (e, e), 0)
            <= lax.broadcasted_iota(jnp.int32, (e, e), 1)).astype(jnp.float32)
    cumincl = lax.dot_general(nblk_row, m_le, (((1,), (0,)), ((), ())),
                              preferred_element_type=jnp.float32)
    gs_row = (cumincl - nblk_row) * bf

    base = gs_row + cum
    pos0 = jnp.sum(jnp.where(oh1, base, 0.0), axis=1, keepdims=True)
    pos1 = jnp.sum(jnp.where(oh2, base, 0.0), axis=1, keepdims=True)
    pos_ref[...] = jnp.concatenate([pos0, pos1], axis=1).astype(jnp.int32)
    w_ref[...] = jnp.concatenate([w1 / denom, w2 / denom], axis=1)

    bid = lax.broadcasted_iota(jnp.int32, (n_blocks, 1), 0).astype(jnp.float32)
    raw = jnp.sum((bid >= cumincl).astype(jnp.float32), axis=1, keepdims=True)
    bexp_ref[...] = jnp.minimum(raw, jnp.float32(e - 1)).astype(jnp.int32)
    nact_ref[...] = cumincl[:, e - 1:e].astype(jnp.int32)


def _grouped_kernel(bexp_ref, nact_ref, xs_ref, w1_ref, w2_ref, sw_ref, ys_ref):
    b = pl.program_id(0)

    @pl.when(b < nact_ref[0])
    def _():
        a = lax.dot_general(xs_ref[...].astype(jnp.bfloat16), w1_ref[0],
                            (((1,), (1,)), ((), ())),
                            preferred_element_type=jnp.float32)
        h = jnp.square(jnp.maximum(a, 0.0)).astype(jnp.bfloat16)
        y = lax.dot_general(h, w2_ref[0], (((1,), (1,)), ((), ())),
                            preferred_element_type=jnp.float32)
        ys_ref[...] = y * sw_ref[0]


def _shared_combine_kernel(x_ref, w1_ref, w2_ref, y0_ref, y1_ref, out_ref):
    a = lax.dot_general(x_ref[...], w1_ref[...], (((1,), (1,)), ((), ())),
                        preferred_element_type=jnp.float32)
    h = jnp.square(jnp.maximum(a, 0.0)).astype(jnp.bfloat16)
    s = lax.dot_general(h, w2_ref[...], (((1,), (1,)), ((), ())),
                        preferred_element_type=jnp.float32)
    out_ref[...] = s + y0_ref[...] + y1_ref[...]


def kernel(hidden_states, gate_weight, e_score_correction_bias, shared_w1,
           shared_w2, expert_w1, expert_w2):
    T, H = hidden_states.shape
    E, I_, _ = expert_w1.shape
    SI = shared_w1.shape[0]
    NB = (T * 2) // _B + E
    S_pad = NB * _B
    SL = H // 128

    x = hidden_states
    pos, wts, bexp2, nact2 = pl.pallas_call(
        functools.partial(_route_kernel, n_blocks=NB),
        out_shape=(
            jax.ShapeDtypeStruct((T, 2), jnp.int32),
            jax.ShapeDtypeStruct((T, 2), jnp.float32),
            jax.ShapeDtypeStruct((NB, 1), jnp.int32),
            jax.ShapeDtypeStruct((1, 1), jnp.int32),
        ),
    )(x, gate_weight, e_score_correction_bias.reshape(1, E))

    pos_flat = pos.T.reshape(-1)
    w_flat = wts.T.reshape(-1)
    x_bf = x.astype(jnp.bfloat16)

    info = plsc.get_sparse_core_info()
    NW = info.num_cores * info.num_subcores
    mesh = plsc.VectorSubcoreMesh(core_axis_name="c", subcore_axis_name="s")

    @functools.partial(
        pl.kernel, mesh=mesh,
        out_type=(
            jax.ShapeDtypeStruct((S_pad, H), jnp.float32),
            jax.ShapeDtypeStruct((S_pad,), jnp.float32),
        ),
        scratch_types=[
            pltpu.VMEM((_CCH, H), jnp.float32),
            pltpu.VMEM((_CCH,), jnp.int32),
            pltpu.VMEM((_CCH,), jnp.int32),
            pltpu.VMEM((_CCH,), jnp.float32),
            pltpu.VMEM((_CCH,), jnp.float32),
            pltpu.SemaphoreType.DMA,
        ],
    )
    def _dispatch(x_hbm, pos_hbm, w_hbm, xs_hbm, sw_hbm,
                  xv, i0, i1, w0, w1v, sem):
        wid = lax.axis_index("s") * info.num_cores + lax.axis_index("c")
        for j in range(_CHUNK // _CCH):
            base = wid * _CHUNK + j * _CCH
            pltpu.sync_copy(x_hbm.at[pl.ds(base, _CCH)], xv)
            pltpu.sync_copy(pos_hbm.at[pl.ds(base, _CCH)], i0)
            pltpu.sync_copy(pos_hbm.at[pl.ds(T + base, _CCH)], i1)
            pltpu.sync_copy(w_hbm.at[pl.ds(base, _CCH)], w0)
            pltpu.sync_copy(w_hbm.at[pl.ds(T + base, _CCH)], w1v)
            pltpu.async_copy(xv, xs_hbm.at[i0], sem).wait()
            pltpu.async_copy(xv, xs_hbm.at[i1], sem).wait()
            pltpu.async_copy(w0, sw_hbm.at[i0], sem).wait()
            pltpu.async_copy(w1v, sw_hbm.at[i1], sem).wait()

    xs2, slot_w = _dispatch(x, pos_flat, w_flat)

    ys = pl.pallas_call(
        _grouped_kernel,
        grid_spec=pltpu.PrefetchScalarGridSpec(
            num_scalar_prefetch=2,
            grid=(NB,),
            in_specs=[
                pl.BlockSpec((_B, H), lambda b, be, na: (b, 0)),
                pl.BlockSpec((1, I_, H), lambda b, be, na: (be[b], 0, 0)),
                pl.BlockSpec((1, H, I_), lambda b, be, na: (be[b], 0, 0)),
                pl.BlockSpec((1, _B, 1), lambda b, be, na: (b, 0, 0)),
            ],
            out_specs=pl.BlockSpec((_B, H), lambda b, be, na: (b, 0)),
        ),
        out_shape=jax.ShapeDtypeStruct((S_pad, H), jnp.float32),
        compiler_params=pltpu.CompilerParams(
            dimension_semantics=("arbitrary",),
        ),
    )(bexp2.reshape(NB), nact2.reshape(1), xs2,
      expert_w1.astype(jnp.bfloat16), expert_w2.astype(jnp.bfloat16),
      slot_w.reshape(NB, _B, 1))

    @functools.partial(
        pl.kernel, mesh=mesh,
        out_type=jax.ShapeDtypeStruct((2 * T, H), jnp.float32),
        scratch_types=[
            pltpu.VMEM((_CCH, H), jnp.float32),
            pltpu.VMEM((_CCH,), jnp.int32),
            pltpu.SemaphoreType.DMA,
        ],
    )
    def _ygather(pos_hbm, ys_hbm, yt_hbm, buf, iv, sem):
        wid = lax.axis_index("s") * info.num_cores + lax.axis_index("c")
        for k in range(2):
            for j in range(_CHUNK // _CCH):
                b2 = k * T + wid * _CHUNK + j * _CCH
                pltpu.sync_copy(pos_hbm.at[pl.ds(b2, _CCH)], iv)
                pltpu.async_copy(ys_hbm.at[iv], buf, sem).wait()
                pltpu.sync_copy(buf, yt_hbm.at[pl.ds(b2, _CCH)])

    yt = _ygather(pos_flat, ys)

    TB = 512
    return pl.pallas_call(
        _shared_combine_kernel,
        grid=(T // TB,),
        in_specs=[
            pl.BlockSpec((TB, H), lambda i: (i, 0)),
            pl.BlockSpec((SI, H), lambda i: (0, 0)),
            pl.BlockSpec((H, SI), lambda i: (0, 0)),
            pl.BlockSpec((TB, H), lambda i: (i, 0)),
            pl.BlockSpec((TB, H), lambda i: (i + T // TB, 0)),
        ],
        out_specs=pl.BlockSpec((TB, H), lambda i: (i, 0)),
        out_shape=jax.ShapeDtypeStruct((T, H), jnp.float32),
    )(x_bf, shared_w1.astype(jnp.bfloat16), shared_w2.astype(jnp.bfloat16),
      yt, yt)


# speedup vs baseline: 1.4940x; 1.1293x over previous
import functools

import jax
import jax.numpy as jnp
from jax import lax
from jax.experimental import pallas as pl
from jax.experimental.pallas import tpu as pltpu
from jax.experimental.pallas import tpu_sc as plsc

_B = 128
_CHUNK = 64
_CCH = 32


def _route_kernel(x_ref, gw_ref, bias_ref, pos_ref, w_ref, bexp_ref, nact_ref,
                  *, n_blocks):
    x = x_ref[...]
    gw = gw_ref[...]
    t, e = x.shape[0], gw.shape[0]
    logits = lax.dot_general(
        x.astype(jnp.bfloat16), gw.astype(jnp.bfloat16), (((1,), (1,)), ((), ())),
        preferred_element_type=jnp.float32)
    scores = jax.nn.sigmoid(logits)
    sfc = scores + bias_ref[...]
    eidx = lax.broadcasted_iota(jnp.int32, (t, e), 1)
    m1 = jnp.max(sfc, axis=1, keepdims=True)
    i1 = jnp.min(jnp.where(sfc == m1, eidx, e), axis=1, keepdims=True)
    oh1 = eidx == i1
    w1 = jnp.sum(jnp.where(oh1, scores, 0.0), axis=1, keepdims=True)
    sfc2 = jnp.where(oh1, -1e30, sfc)
    m2 = jnp.max(sfc2, axis=1, keepdims=True)
    i2 = jnp.min(jnp.where(sfc2 == m2, eidx, e), axis=1, keepdims=True)
    oh2 = eidx == i2
    w2 = jnp.sum(jnp.where(oh2, scores, 0.0), axis=1, keepdims=True)
    denom = w1 + w2 + 1e-20

    oh = (oh1 | oh2).astype(jnp.float32)
    C = 256
    lt = (lax.broadcasted_iota(jnp.int32, (C, C), 0)
          > lax.broadcasted_iota(jnp.int32, (C, C), 1)).astype(jnp.float32)
    run = jnp.zeros((1, e), jnp.float32)
    cums = []
    for c in range(t // C):
        ohc = oh[c * C:(c + 1) * C]
        exc = lax.dot_general(lt, ohc, (((1,), (0,)), ((), ())),
                              preferred_element_type=jnp.float32) + run
        cums.append(exc)
        run = run + jnp.sum(ohc, axis=0, keepdims=True)
    cum = jnp.concatenate(cums, axis=0)
    counts = run

    bf = jnp.float32(_B)
    nblk_row = jnp.floor((counts + (bf - 1.0)) / bf)
    m_le = (lax.broadcasted_iota(jnp.int32, (e, e), 0)
            <= lax.broadcasted_iota(jnp.int32, (e, e), 1)).astype(jnp.float32)
    cumincl = lax.dot_general(nblk_row, m_le, (((1,), (0,)), ((), ())),
                              preferred_element_type=jnp.float32)
    gs_row = (cumincl - nblk_row) * bf

    base = gs_row + cum
    pos0 = jnp.sum(jnp.where(oh1, base, 0.0), axis=1, keepdims=True)
    pos1 = jnp.sum(jnp.where(oh2, base, 0.0), axis=1, keepdims=True)
    pos_ref[...] = jnp.concatenate([pos0, pos1], axis=1).astype(jnp.int32)
    w_ref[...] = jnp.concatenate([w1 / denom, w2 / denom], axis=1)

    bid = lax.broadcasted_iota(jnp.int32, (n_blocks, 1), 0).astype(jnp.float32)
    raw = jnp.sum((bid >= cumincl).astype(jnp.float32), axis=1, keepdims=True)
    bexp_ref[...] = jnp.minimum(raw, jnp.float32(e - 1)).astype(jnp.int32)
    nact_ref[...] = cumincl[:, e - 1:e].astype(jnp.int32)


def _grouped_kernel(bexp_ref, nact_ref, xs_ref, w1_ref, w2_ref, sw_ref, ys_ref):
    b = pl.program_id(0)

    @pl.when(b < nact_ref[0])
    def _():
        a = lax.dot_general(xs_ref[...].astype(jnp.bfloat16),
                            w1_ref[0].astype(jnp.bfloat16),
                            (((1,), (1,)), ((), ())),
                            preferred_element_type=jnp.float32)
        h = jnp.square(jnp.maximum(a, 0.0)).astype(jnp.bfloat16)
        y = lax.dot_general(h, w2_ref[0].astype(jnp.bfloat16),
                            (((1,), (1,)), ((), ())),
                            preferred_element_type=jnp.float32)
        ys_ref[...] = y * sw_ref[0]


def _shared_combine_kernel(x_ref, w1_ref, w2_ref, y0_ref, y1_ref, out_ref):
    a = lax.dot_general(x_ref[...].astype(jnp.bfloat16),
                        w1_ref[...].astype(jnp.bfloat16),
                        (((1,), (1,)), ((), ())),
                        preferred_element_type=jnp.float32)
    h = jnp.square(jnp.maximum(a, 0.0)).astype(jnp.bfloat16)
    s = lax.dot_general(h, w2_ref[...].astype(jnp.bfloat16),
                        (((1,), (1,)), ((), ())),
                        preferred_element_type=jnp.float32)
    out_ref[...] = s + y0_ref[...] + y1_ref[...]


def kernel(hidden_states, gate_weight, e_score_correction_bias, shared_w1,
           shared_w2, expert_w1, expert_w2):
    T, H = hidden_states.shape
    E, I_, _ = expert_w1.shape
    SI = shared_w1.shape[0]
    NB = (T * 2) // _B + E
    S_pad = NB * _B
    SL = H // 128

    x = hidden_states
    pos, wts, bexp2, nact2 = pl.pallas_call(
        functools.partial(_route_kernel, n_blocks=NB),
        out_shape=(
            jax.ShapeDtypeStruct((T, 2), jnp.int32),
            jax.ShapeDtypeStruct((T, 2), jnp.float32),
            jax.ShapeDtypeStruct((NB, 1), jnp.int32),
            jax.ShapeDtypeStruct((1, 1), jnp.int32),
        ),
    )(x, gate_weight, e_score_correction_bias.reshape(1, E))

    pos_flat = pos.T.reshape(-1)
    w_flat = wts.T.reshape(-1)

    info = plsc.get_sparse_core_info()
    NW = info.num_cores * info.num_subcores
    mesh = plsc.VectorSubcoreMesh(core_axis_name="c", subcore_axis_name="s")

    SUB = 16
    NSUB = _CHUNK // SUB

    @functools.partial(
        pl.kernel, mesh=mesh,
        out_type=(
            jax.ShapeDtypeStruct((S_pad, H), jnp.float32),
            jax.ShapeDtypeStruct((S_pad,), jnp.float32),
        ),
        scratch_types=[
            pltpu.VMEM((SUB, H), jnp.float32),
            pltpu.VMEM((SUB, H), jnp.float32),
            pltpu.VMEM((SUB,), jnp.int32),
            pltpu.VMEM((SUB,), jnp.int32),
            pltpu.VMEM((SUB,), jnp.int32),
            pltpu.VMEM((SUB,), jnp.int32),
            pltpu.VMEM((SUB,), jnp.float32),
            pltpu.VMEM((SUB,), jnp.float32),
            pltpu.VMEM((SUB,), jnp.float32),
            pltpu.VMEM((SUB,), jnp.float32),
            pltpu.SemaphoreType.DMA,
            pltpu.SemaphoreType.DMA,
        ],
    )
    def _dispatch(x_hbm, pos_hbm, w_hbm, xs_hbm, sw_hbm,
                  xv0, xv1, i0a, i0b, i1a, i1b, w0a, w0b, w1a, w1b, s0, s1):
        wid = lax.axis_index("s") * info.num_cores + lax.axis_index("c")
        xv, i0, i1 = (xv0, xv1), (i0a, i0b), (i1a, i1b)
        w0, w1 = (w0a, w0b), (w1a, w1b)
        sems = (s0, s1)
        pend = [None, None]
        for j in range(NSUB):
            sl = j % 2
            if pend[sl]:
                for hnd in pend[sl]:
                    hnd.wait()
            base = wid * _CHUNK + j * SUB
            pltpu.sync_copy(x_hbm.at[pl.ds(base, SUB)], xv[sl])
            pltpu.sync_copy(pos_hbm.at[pl.ds(base, SUB)], i0[sl])
            pltpu.sync_copy(pos_hbm.at[pl.ds(T + base, SUB)], i1[sl])
            pltpu.sync_copy(w_hbm.at[pl.ds(base, SUB)], w0[sl])
            pltpu.sync_copy(w_hbm.at[pl.ds(T + base, SUB)], w1[sl])
            pend[sl] = [
                pltpu.async_copy(xv[sl], xs_hbm.at[i0[sl]], sems[sl]),
                pltpu.async_copy(xv[sl], xs_hbm.at[i1[sl]], sems[sl]),
                pltpu.async_copy(w0[sl], sw_hbm.at[i0[sl]], sems[sl]),
                pltpu.async_copy(w1[sl], sw_hbm.at[i1[sl]], sems[sl]),
            ]
        for p in pend:
            if p:
                for hnd in p:
                    hnd.wait()

    xs2, slot_w = _dispatch(x, pos_flat, w_flat)

    ys = pl.pallas_call(
        _grouped_kernel,
        grid_spec=pltpu.PrefetchScalarGridSpec(
            num_scalar_prefetch=2,
            grid=(NB,),
            in_specs=[
                pl.BlockSpec((_B, H), lambda b, be, na: (b, 0)),
                pl.BlockSpec((1, I_, H), lambda b, be, na: (be[b], 0, 0)),
                pl.BlockSpec((1, H, I_), lambda b, be, na: (be[b], 0, 0)),
                pl.BlockSpec((1, _B, 1), lambda b, be, na: (b, 0, 0)),
            ],
            out_specs=pl.BlockSpec((_B, H), lambda b, be, na: (b, 0)),
        ),
        out_shape=jax.ShapeDtypeStruct((S_pad, H), jnp.float32),
        compiler_params=pltpu.CompilerParams(
            dimension_semantics=("arbitrary",),
        ),
    )(bexp2.reshape(NB), nact2.reshape(1), xs2,
      expert_w1, expert_w2, slot_w.reshape(NB, _B, 1))

    @functools.partial(
        pl.kernel, mesh=mesh,
        out_type=jax.ShapeDtypeStruct((2 * T, H), jnp.float32),
        scratch_types=[
            pltpu.VMEM((SUB, H), jnp.float32),
            pltpu.VMEM((SUB, H), jnp.float32),
            pltpu.VMEM((SUB,), jnp.int32),
            pltpu.VMEM((SUB,), jnp.int32),
            pltpu.SemaphoreType.DMA,
            pltpu.SemaphoreType.DMA,
            pltpu.SemaphoreType.DMA,
            pltpu.SemaphoreType.DMA,
        ],
    )
    def _ygather(pos_hbm, ys_hbm, yt_hbm, b0, b1, iv0, iv1, g0, g1, st0, st1):
        wid = lax.axis_index("s") * info.num_cores + lax.axis_index("c")
        bufs, ivs = (b0, b1), (iv0, iv1)
        gsem, ssem = (g0, g1), (st0, st1)
        pend = [None, None]
        for k in range(2):
            for j in range(NSUB):
                jj = k * NSUB + j
                sl = jj % 2
                if pend[sl]:
                    pend[sl].wait()
                b2 = k * T + wid * _CHUNK + j * SUB
                pltpu.sync_copy(pos_hbm.at[pl.ds(b2, SUB)], ivs[sl])
                pltpu.async_copy(ys_hbm.at[ivs[sl]], bufs[sl], gsem[sl]).wait()
                pend[sl] = pltpu.async_copy(bufs[sl], yt_hbm.at[pl.ds(b2, SUB)],
                                            ssem[sl])
        for p in pend:
            if p:
                p.wait()

    yt = _ygather(pos_flat, ys)

    TB = 256
    return pl.pallas_call(
        _shared_combine_kernel,
        grid=(T // TB,),
        in_specs=[
            pl.BlockSpec((TB, H), lambda i: (i, 0)),
            pl.BlockSpec((SI, H), lambda i: (0, 0)),
            pl.BlockSpec((H, SI), lambda i: (0, 0)),
            pl.BlockSpec((TB, H), lambda i: (i, 0)),
            pl.BlockSpec((TB, H), lambda i: (i + T // TB, 0)),
        ],
        out_specs=pl.BlockSpec((TB, H), lambda i: (i, 0)),
        out_shape=jax.ShapeDtypeStruct((T, H), jnp.float32),
    )(x, shared_w1, shared_w2, yt, yt)


# speedup vs baseline: 1.8009x; 1.2054x over previous
import functools

import jax
import jax.numpy as jnp
from jax import lax
from jax.experimental import pallas as pl
from jax.experimental.pallas import tpu as pltpu
from jax.experimental.pallas import tpu_sc as plsc

_B = 256
_CHUNK = 64
_CCH = 32


def _route_kernel(x_ref, gw_ref, bias_ref, pos_ref, w_ref, bexp_ref, nact_ref,
                  *, n_blocks):
    x = x_ref[...]
    gw = gw_ref[...]
    t, e = x.shape[0], gw.shape[0]
    logits = lax.dot_general(
        x.astype(jnp.bfloat16), gw.astype(jnp.bfloat16), (((1,), (1,)), ((), ())),
        preferred_element_type=jnp.float32)
    scores = jax.nn.sigmoid(logits)
    sfc = scores + bias_ref[...]
    eidx = lax.broadcasted_iota(jnp.int32, (t, e), 1)
    m1 = jnp.max(sfc, axis=1, keepdims=True)
    i1 = jnp.min(jnp.where(sfc == m1, eidx, e), axis=1, keepdims=True)
    oh1 = eidx == i1
    w1 = jnp.sum(jnp.where(oh1, scores, 0.0), axis=1, keepdims=True)
    sfc2 = jnp.where(oh1, -1e30, sfc)
    m2 = jnp.max(sfc2, axis=1, keepdims=True)
    i2 = jnp.min(jnp.where(sfc2 == m2, eidx, e), axis=1, keepdims=True)
    oh2 = eidx == i2
    w2 = jnp.sum(jnp.where(oh2, scores, 0.0), axis=1, keepdims=True)
    denom = w1 + w2 + 1e-20

    oh = (oh1 | oh2).astype(jnp.float32)
    C = 256
    lt = (lax.broadcasted_iota(jnp.int32, (C, C), 0)
          > lax.broadcasted_iota(jnp.int32, (C, C), 1)).astype(jnp.float32)
    run = jnp.zeros((1, e), jnp.float32)
    cums = []
    for c in range(t // C):
        ohc = oh[c * C:(c + 1) * C]
        exc = lax.dot_general(lt, ohc, (((1,), (0,)), ((), ())),
                              preferred_element_type=jnp.float32) + run
        cums.append(exc)
        run = run + jnp.sum(ohc, axis=0, keepdims=True)
    cum = jnp.concatenate(cums, axis=0)
    counts = run

    bf = jnp.float32(_B)
    nblk_row = jnp.floor((counts + (bf - 1.0)) / bf)
    m_le = (lax.broadcasted_iota(jnp.int32, (e, e), 0)
            <= lax.broadcasted_iota(jnp.int32, (e, e), 1)).astype(jnp.float32)
    cumincl = lax.dot_general(nblk_row, m_le, (((1,), (0,)), ((), ())),
                              preferred_element_type=jnp.float32)
    gs_row = (cumincl - nblk_row) * bf

    base = gs_row + cum
    pos0 = jnp.sum(jnp.where(oh1, base, 0.0), axis=1, keepdims=True)
    pos1 = jnp.sum(jnp.where(oh2, base, 0.0), axis=1, keepdims=True)
    pos_ref[...] = jnp.concatenate([pos0, pos1], axis=1).astype(jnp.int32)
    w_ref[...] = jnp.concatenate([w1 / denom, w2 / denom], axis=1)

    bid = lax.broadcasted_iota(jnp.int32, (n_blocks, 1), 0).astype(jnp.float32)
    raw = jnp.sum((bid >= cumincl).astype(jnp.float32), axis=1, keepdims=True)
    bexp_ref[...] = jnp.minimum(raw, jnp.float32(e - 1)).astype(jnp.int32)
    nact_ref[...] = cumincl[:, e - 1:e].astype(jnp.int32)


def _grouped_kernel(bexp_ref, nact_ref, xs_ref, w1_ref, w2_ref, sw_ref, ys_ref):
    b = pl.program_id(0)

    @pl.when(b < nact_ref[0])
    def _():
        a = lax.dot_general(xs_ref[...].astype(jnp.bfloat16),
                            w1_ref[0].astype(jnp.bfloat16),
                            (((1,), (1,)), ((), ())),
                            preferred_element_type=jnp.float32)
        h = jnp.square(jnp.maximum(a, 0.0)).astype(jnp.bfloat16)
        y = lax.dot_general(h, w2_ref[0].astype(jnp.bfloat16),
                            (((1,), (1,)), ((), ())),
                            preferred_element_type=jnp.float32)
        ys_ref[...] = y * sw_ref[0]


def _shared_combine_kernel(x_ref, w1_ref, w2_ref, y0_ref, y1_ref, out_ref):
    a = lax.dot_general(x_ref[...].astype(jnp.bfloat16),
                        w1_ref[...].astype(jnp.bfloat16),
                        (((1,), (1,)), ((), ())),
                        preferred_element_type=jnp.float32)
    h = jnp.square(jnp.maximum(a, 0.0)).astype(jnp.bfloat16)
    s = lax.dot_general(h, w2_ref[...].astype(jnp.bfloat16),
                        (((1,), (1,)), ((), ())),
                        preferred_element_type=jnp.float32)
    out_ref[...] = s + y0_ref[...] + y1_ref[...]


def kernel(hidden_states, gate_weight, e_score_correction_bias, shared_w1,
           shared_w2, expert_w1, expert_w2):
    T, H = hidden_states.shape
    E, I_, _ = expert_w1.shape
    SI = shared_w1.shape[0]
    NB = (T * 2) // _B + E
    S_pad = NB * _B
    SL = H // 128

    x = hidden_states
    pos, wts, bexp2, nact2 = pl.pallas_call(
        functools.partial(_route_kernel, n_blocks=NB),
        out_shape=(
            jax.ShapeDtypeStruct((T, 2), jnp.int32),
            jax.ShapeDtypeStruct((T, 2), jnp.float32),
            jax.ShapeDtypeStruct((NB, 1), jnp.int32),
            jax.ShapeDtypeStruct((1, 1), jnp.int32),
        ),
    )(x, gate_weight, e_score_correction_bias.reshape(1, E))

    pos_flat = pos.T.reshape(-1)
    w_flat = wts.T.reshape(-1)

    info = plsc.get_sparse_core_info()
    NW = info.num_cores * info.num_subcores
    mesh = plsc.VectorSubcoreMesh(core_axis_name="c", subcore_axis_name="s")

    SUB = 16
    NSUB = _CHUNK // SUB

    @functools.partial(
        pl.kernel, mesh=mesh,
        out_type=(
            jax.ShapeDtypeStruct((S_pad, H), jnp.float32),
            jax.ShapeDtypeStruct((S_pad,), jnp.float32),
        ),
        scratch_types=[
            pltpu.VMEM((SUB, H), jnp.float32),
            pltpu.VMEM((SUB, H), jnp.float32),
            pltpu.VMEM((SUB,), jnp.int32),
            pltpu.VMEM((SUB,), jnp.int32),
            pltpu.VMEM((SUB,), jnp.int32),
            pltpu.VMEM((SUB,), jnp.int32),
            pltpu.VMEM((SUB,), jnp.float32),
            pltpu.VMEM((SUB,), jnp.float32),
            pltpu.VMEM((SUB,), jnp.float32),
            pltpu.VMEM((SUB,), jnp.float32),
            pltpu.SemaphoreType.DMA,
            pltpu.SemaphoreType.DMA,
        ],
    )
    def _dispatch(x_hbm, pos_hbm, w_hbm, xs_hbm, sw_hbm,
                  xv0, xv1, i0a, i0b, i1a, i1b, w0a, w0b, w1a, w1b, s0, s1):
        wid = lax.axis_index("s") * info.num_cores + lax.axis_index("c")
        xv, i0, i1 = (xv0, xv1), (i0a, i0b), (i1a, i1b)
        w0, w1 = (w0a, w0b), (w1a, w1b)
        sems = (s0, s1)
        pend = [None, None]
        for j in range(NSUB):
            sl = j % 2
            if pend[sl]:
                for hnd in pend[sl]:
                    hnd.wait()
            base = wid * _CHUNK + j * SUB
            pltpu.sync_copy(x_hbm.at[pl.ds(base, SUB)], xv[sl])
            pltpu.sync_copy(pos_hbm.at[pl.ds(base, SUB)], i0[sl])
            pltpu.sync_copy(pos_hbm.at[pl.ds(T + base, SUB)], i1[sl])
            pltpu.sync_copy(w_hbm.at[pl.ds(base, SUB)], w0[sl])
            pltpu.sync_copy(w_hbm.at[pl.ds(T + base, SUB)], w1[sl])
            pend[sl] = [
                pltpu.async_copy(xv[sl], xs_hbm.at[i0[sl]], sems[sl]),
                pltpu.async_copy(xv[sl], xs_hbm.at[i1[sl]], sems[sl]),
                pltpu.async_copy(w0[sl], sw_hbm.at[i0[sl]], sems[sl]),
                pltpu.async_copy(w1[sl], sw_hbm.at[i1[sl]], sems[sl]),
            ]
        for p in pend:
            if p:
                for hnd in p:
                    hnd.wait()

    xs2, slot_w = _dispatch(x, pos_flat, w_flat)

    ys = pl.pallas_call(
        _grouped_kernel,
        grid_spec=pltpu.PrefetchScalarGridSpec(
            num_scalar_prefetch=2,
            grid=(NB,),
            in_specs=[
                pl.BlockSpec((_B, H), lambda b, be, na: (b, 0)),
                pl.BlockSpec((1, I_, H), lambda b, be, na: (be[b], 0, 0)),
                pl.BlockSpec((1, H, I_), lambda b, be, na: (be[b], 0, 0)),
                pl.BlockSpec((1, _B, 1), lambda b, be, na: (b, 0, 0)),
            ],
            out_specs=pl.BlockSpec((_B, H), lambda b, be, na: (b, 0)),
        ),
        out_shape=jax.ShapeDtypeStruct((S_pad, H), jnp.float32),
        compiler_params=pltpu.CompilerParams(
            dimension_semantics=("arbitrary",),
        ),
    )(bexp2.reshape(NB), nact2.reshape(1), xs2,
      expert_w1, expert_w2, slot_w.reshape(NB, _B, 1))

    @functools.partial(
        pl.kernel, mesh=mesh,
        out_type=jax.ShapeDtypeStruct((2 * T, H), jnp.float32),
        scratch_types=[
            pltpu.VMEM((SUB, H), jnp.float32),
            pltpu.VMEM((SUB, H), jnp.float32),
            pltpu.VMEM((SUB,), jnp.int32),
            pltpu.VMEM((SUB,), jnp.int32),
            pltpu.SemaphoreType.DMA,
            pltpu.SemaphoreType.DMA,
            pltpu.SemaphoreType.DMA,
            pltpu.SemaphoreType.DMA,
        ],
    )
    def _ygather(pos_hbm, ys_hbm, yt_hbm, b0, b1, iv0, iv1, g0, g1, st0, st1):
        wid = lax.axis_index("s") * info.num_cores + lax.axis_index("c")
        bufs, ivs = (b0, b1), (iv0, iv1)
        gsem, ssem = (g0, g1), (st0, st1)
        pend = [None, None]
        for k in range(2):
            for j in range(NSUB):
                jj = k * NSUB + j
                sl = jj % 2
                if pend[sl]:
                    pend[sl].wait()
                b2 = k * T + wid * _CHUNK + j * SUB
                pltpu.sync_copy(pos_hbm.at[pl.ds(b2, SUB)], ivs[sl])
                pltpu.async_copy(ys_hbm.at[ivs[sl]], bufs[sl], gsem[sl]).wait()
                pend[sl] = pltpu.async_copy(bufs[sl], yt_hbm.at[pl.ds(b2, SUB)],
                                            ssem[sl])
        for p in pend:
            if p:
                p.wait()

    yt = _ygather(pos_flat, ys)

    TB = 256
    return pl.pallas_call(
        _shared_combine_kernel,
        grid=(T // TB,),
        in_specs=[
            pl.BlockSpec((TB, H), lambda i: (i, 0)),
            pl.BlockSpec((SI, H), lambda i: (0, 0)),
            pl.BlockSpec((H, SI), lambda i: (0, 0)),
            pl.BlockSpec((TB, H), lambda i: (i, 0)),
            pl.BlockSpec((TB, H), lambda i: (i + T // TB, 0)),
        ],
        out_specs=pl.BlockSpec((TB, H), lambda i: (i, 0)),
        out_shape=jax.ShapeDtypeStruct((T, H), jnp.float32),
    )(x, shared_w1, shared_w2, yt, yt)
